# baseline (speedup 1.0000x reference)
import functools

import jax
import jax.numpy as jnp
from jax import lax
from jax.experimental import pallas as pl
from jax.experimental.pallas import tpu as pltpu
from jax.experimental.pallas import tpu_sc as plsc

B = 4096
D = 64
SEQ = 50
NC = 2
NS = 16
NW = NC * NS

NSPLIT = 2
GB = B // NSPLIT
IPW = GB // NW
UPW = B // NW
SPW = SEQ * GB // NW
SCHUNK = 640
NCHUNK = SPW // SCHUNK

BB = 256
NB1 = GB // BB
BB2 = 512
NB2 = B // BB2


_SC_PARAMS = pltpu.CompilerParams(use_tc_tiling_on_sc=False)


def _sc_user(user_pairs, uhalf):
    mesh = plsc.VectorSubcoreMesh(core_axis_name="c", subcore_axis_name="s")

    @functools.partial(
        pl.kernel,
        mesh=mesh,
        out_type=jax.ShapeDtypeStruct((B, 2 * D), jnp.float32),
        scratch_types=[
            pltpu.VMEM((UPW,), jnp.int32),
            pltpu.VMEM((UPW, 2 * D), jnp.float32),
            pltpu.SemaphoreType.DMA,
        ],
    )
    def k(tab, idx_hbm, out, idx_v, rows_v, sem):
        wid = lax.axis_index("s") * NC + lax.axis_index("c")
        base = wid * UPW
        pltpu.sync_copy(idx_hbm.at[pl.ds(base, UPW)], idx_v)
        pltpu.async_copy(tab.at[idx_v], rows_v, sem).wait()
        pltpu.sync_copy(rows_v, out.at[pl.ds(base, UPW)])

    return k(user_pairs, uhalf)


def _sc_group(item_table, iidx, sidx3):
    mesh = plsc.VectorSubcoreMesh(core_axis_name="c", subcore_axis_name="s")

    @functools.partial(
        pl.kernel,
        mesh=mesh,
        out_type=(
            jax.ShapeDtypeStruct((GB, D), jnp.float32),
            jax.ShapeDtypeStruct((SEQ * GB, D), jnp.float32),
        ),
        scratch_types=[
            pltpu.VMEM((IPW,), jnp.int32),
            pltpu.VMEM((IPW, D), jnp.float32),
            pltpu.VMEM((NCHUNK, SCHUNK), jnp.int32),
            pltpu.VMEM((SCHUNK, D), jnp.float32),
            pltpu.VMEM((SCHUNK, D), jnp.float32),
            pltpu.SemaphoreType.DMA,
            pltpu.SemaphoreType.DMA,
        ],
        compiler_params=_SC_PARAMS,
    )
    def k(tab, iref, sref, i_out, s_out,
          idx_v, rows_v, sidx_v, srows_a, srows_b, sem_a, sem_b):
        wid = lax.axis_index("s") * NC + lax.axis_index("c")
        base = wid * IPW
        pltpu.sync_copy(iref.at[pl.ds(base, IPW)], idx_v)
        pltpu.async_copy(tab.at[idx_v], rows_v, sem_a).wait()
        pltpu.sync_copy(rows_v, i_out.at[pl.ds(base, IPW)])
        pltpu.sync_copy(sref.at[wid], sidx_v)
        sbase = wid * SPW
        bufs = (srows_a, srows_b)
        sems = (sem_a, sem_b)
        copies = [
            pltpu.async_copy(tab.at[sidx_v.at[c]], bufs[c % 2], sems[c % 2])
            for c in range(2)
        ]
        for c in range(NCHUNK):
            copies[c % 2].wait()
            pltpu.sync_copy(bufs[c % 2],
                            s_out.at[pl.ds(sbase + c * SCHUNK, SCHUNK)])
            if c + 2 < NCHUNK:
                copies[c % 2] = pltpu.async_copy(
                    tab.at[sidx_v.at[c + 2]], bufs[c % 2], sems[c % 2])

    return k(item_table, iidx, sidx3)


def _dot(a, b):
    return lax.dot_general(a, b, (((1,), (0,)), ((), ())),
                           preferred_element_type=jnp.float32,
                           precision=lax.Precision.HIGHEST)


def _tc1_body(irows, seq3, maskf, g_i, a_i, c_i, cl_i, num, text,
              g_tab, a_tab, c_tab, cl_tab, pos,
              W_num, b_num, g_num, be_num,
              W_text, b_text, g_text, be_text,
              W_seq, b_seq, W_a1, b_a1, W_a2, b_a2,
              W_u1, b_u1, W_i1, b_i1, W_i2, b_i2,
              pu_ref, ivn_ref):
    relu = lambda x: jnp.maximum(x, 0.0)

    def onehot_embed(idx_ref, tab_ref, n):
        iot = lax.broadcasted_iota(jnp.int32, (BB, n), 1)
        oh = (iot == idx_ref[...]).astype(jnp.float32)
        return _dot(oh, tab_ref[...])

    g_e = onehot_embed(g_i, g_tab, 3)
    a_e = onehot_embed(a_i, a_tab, 10)
    c_e = onehot_embed(c_i, c_tab, 1000)
    cl_e = onehot_embed(cl_i, cl_tab, 100)

    num_proj = relu((_dot(num[...], W_num[...]) + b_num[...]) * g_num[...]
                    + be_num[...])
    text_proj = relu((_dot(text[...], W_text[...]) + b_text[...])
                     * g_text[...] + be_text[...])

    wa1 = W_a1[...]
    wa2 = W_a2[...]
    ba1 = b_a1[...]
    ba2 = b_a2[...]
    acc_sum = jnp.zeros((BB, D), jnp.float32)
    acc_max = jnp.full((BB, D), -jnp.inf, jnp.float32)
    m = jnp.full((BB, 1), -jnp.inf, jnp.float32)
    l = jnp.zeros((BB, 1), jnp.float32)
    acc_att = jnp.zeros((BB, D), jnp.float32)
    for s in range(SEQ):
        mc = maskf[:, s:s + 1]
        xs = (seq3[s] + pos[s:s + 1, :]) * mc
        acc_sum = acc_sum + xs
        acc_max = jnp.maximum(acc_max, xs)
        h = relu(_dot(xs, wa1) + ba1)
        lg = (_dot(h, wa2) + ba2) * mc - 1e9 * (1.0 - mc)
        nm = jnp.maximum(m, lg)
        sc = jnp.exp(m - nm)
        p = jnp.exp(lg - nm)
        l = l * sc + p
        acc_att = acc_att * sc + p * xs
        m = nm
    valid = jnp.sum(maskf[...], axis=1, keepdims=True)
    mean_p = acc_sum / (valid + 1e-8)
    att_p = acc_att / l

    ws = W_seq[...]
    seq_embed = relu(_dot(mean_p, ws[0:D]) + _dot(acc_max, ws[D:2 * D])
                     + _dot(att_p, ws[2 * D:3 * D]) + b_seq[...])

    wu1 = W_u1[...]
    pu = (_dot(g_e, wu1[D:2 * D]) + _dot(a_e, wu1[2 * D:3 * D])
          + _dot(c_e, wu1[3 * D:4 * D]) + _dot(cl_e, wu1[4 * D:5 * D])
          + _dot(num_proj, wu1[5 * D:6 * D])
          + _dot(seq_embed, wu1[6 * D:7 * D]) + b_u1[...])

    wi1 = W_i1[...]
    iv = relu(_dot(irows[...], wi1[0:D]) + _dot(text_proj, wi1[D:2 * D])
              + b_i1[...])
    iv = relu(_dot(iv, W_i2[...]) + b_i2[...])
    inn = jnp.maximum(jnp.sqrt(jnp.sum(iv * iv, axis=1, keepdims=True)),
                      1e-12)
    pu_ref[...] = pu
    ivn_ref[...] = iv / inn


def _tc2_body(upairs, par, pu, ivn, W_u1, W_u2, b_u2, out_ref):
    relu = lambda x: jnp.maximum(x, 0.0)
    p = par[...]
    urows = upairs[:, 0:D] * (1.0 - p) + upairs[:, D:2 * D] * p
    u = relu(pu[...] + _dot(urows, W_u1[0:D]))
    u2 = relu(_dot(u, W_u2[...]) + b_u2[...])
    un = jnp.maximum(jnp.sqrt(jnp.sum(u2 * u2, axis=1, keepdims=True)),
                     1e-12)
    out_ref[...] = jnp.sum(u2 * ivn[...], axis=1, keepdims=True) / un


def _row_spec(rows, cols):
    return pl.BlockSpec((rows, cols), lambda i: (i, 0))


def _full_spec(shape):
    nd = len(shape)
    return pl.BlockSpec(shape, lambda i: (0,) * nd)


def _tc1_kwargs():
    in_specs = [
        _row_spec(BB, D),
        pl.BlockSpec((SEQ, BB, D), lambda i: (0, i, 0)),
        _row_spec(BB, SEQ),
        _row_spec(BB, 1), _row_spec(BB, 1),
        _row_spec(BB, 1), _row_spec(BB, 1),
        _row_spec(BB, 16),
        _row_spec(BB, 128),
        _full_spec((3, D)), _full_spec((10, D)), _full_spec((1000, D)),
        _full_spec((100, D)), _full_spec((SEQ, D)),
        _full_spec((16, D)), _full_spec((1, D)), _full_spec((1, D)),
        _full_spec((1, D)),
        _full_spec((128, D)), _full_spec((1, D)), _full_spec((1, D)),
        _full_spec((1, D)),
        _full_spec((3 * D, D)), _full_spec((1, D)),
        _full_spec((D, D // 2)), _full_spec((1, D // 2)),
        _full_spec((D // 2, 1)), _full_spec((1, 1)),
        _full_spec((7 * D, 128)), _full_spec((1, 128)),
        _full_spec((2 * D, 128)), _full_spec((1, 128)),
        _full_spec((128, D)), _full_spec((1, D)),
    ]
    return dict(
        grid=(NB1,),
        in_specs=in_specs,
        out_specs=(_row_spec(BB, 128), _row_spec(BB, D)),
        out_shape=(jax.ShapeDtypeStruct((GB, 128), jnp.float32),
                   jax.ShapeDtypeStruct((GB, D), jnp.float32)),
        compiler_params=pltpu.CompilerParams(
            dimension_semantics=("arbitrary",)),
    )


def _tc2_kwargs():
    in_specs = [
        _row_spec(BB2, 2 * D),
        _row_spec(BB2, 1),
        _row_spec(BB2, 128),
        _row_spec(BB2, D),
        _full_spec((7 * D, 128)),
        _full_spec((128, D)),
        _full_spec((1, D)),
    ]
    return dict(
        grid=(NB2,),
        in_specs=in_specs,
        out_specs=_row_spec(BB2, 1),
        out_shape=jax.ShapeDtypeStruct((B, 1), jnp.float32),
        compiler_params=pltpu.CompilerParams(
            dimension_semantics=("arbitrary",)),
    )


def _group_args(g, i_rows, s_rows, sequence_mask, gender, age_range, city,
                cluster_id, user_numeric, item_text_feat, gender_table,
                age_table, city_table, cluster_table, position_table,
                W_num, b_num, g_num, be_num, W_text, b_text, g_text, be_text,
                W_seq, b_seq, W_a1, b_a1, W_a2, b_a2, W_u1, b_u1,
                W_i1, b_i1, W_i2, b_i2):
    r1 = lambda v: v.reshape(1, -1)
    k = 1.0 / jnp.sqrt(jnp.float32(1.0 + 1e-5))
    sl = slice(g * GB, (g + 1) * GB)
    col = lambda v: v[sl].astype(jnp.int32).reshape(GB, 1)
    return (
        i_rows, s_rows.reshape(SEQ, GB, D),
        sequence_mask[sl].astype(jnp.float32),
        col(gender), col(age_range), col(city), col(cluster_id),
        user_numeric[sl], item_text_feat[sl],
        gender_table, age_table, city_table, cluster_table, position_table,
        W_num, r1(b_num), r1(g_num) * k, r1(be_num),
        W_text, r1(b_text), r1(g_text) * k, r1(be_text),
        W_seq, r1(b_seq), W_a1, r1(b_a1), W_a2, r1(b_a2),
        W_u1, r1(b_u1), W_i1, r1(b_i1), W_i2, r1(b_i2),
    )


def kernel(user_id, item_id, gender, age_range, city, cluster_id,
           user_numeric, item_text_feat, user_sequence, sequence_mask,
           user_table, gender_table, age_table, city_table, cluster_table,
           item_table, position_table, W_num, b_num, g_num, be_num,
           W_text, b_text, g_text, be_text, W_seq, b_seq, W_a1, b_a1,
           W_a2, b_a2, W_u1, b_u1, W_u2, b_u2, W_i1, b_i1, W_i2, b_i2):
    uidx = user_id.astype(jnp.int32)
    iidx = item_id.astype(jnp.int32)
    nrows = user_table.shape[0]
    u_par = (uidx % 2).astype(jnp.float32).reshape(B, 1)
    pus, ivns = [], []
    i_rows0 = None
    for g in range(NSPLIT):
        sl = slice(g * GB, (g + 1) * GB)
        sidx3 = user_sequence[sl].astype(jnp.int32).T.reshape(
            NW, NCHUNK, SCHUNK)
        i_rows, s_rows = _sc_group(item_table, iidx[sl], sidx3)
        if g == 0:
            i_rows0 = i_rows
        args = _group_args(g, i_rows, s_rows, sequence_mask, gender,
                           age_range, city, cluster_id, user_numeric,
                           item_text_feat, gender_table, age_table,
                           city_table, cluster_table, position_table,
                           W_num, b_num, g_num, be_num, W_text, b_text,
                           g_text, be_text, W_seq, b_seq, W_a1, b_a1,
                           W_a2, b_a2, W_u1, b_u1, W_i1, b_i1, W_i2, b_i2)
        pu, ivn = pl.pallas_call(_tc1_body, **_tc1_kwargs())(*args)
        pus.append(pu)
        ivns.append(ivn)
    pu = jnp.concatenate(pus, axis=0)
    ivn = jnp.concatenate(ivns, axis=0)
    ut_b, _ = lax.optimization_barrier((user_table, i_rows0))
    u_pairs = _sc_user(ut_b.reshape(nrows // 2, 2 * D), uidx // 2)
    out = pl.pallas_call(_tc2_body, **_tc2_kwargs())(
        u_pairs, u_par, pu, ivn, W_u1, W_u2, b_u2.reshape(1, -1))
    return out.reshape(B)


# speedup vs baseline: 1.1653x; 1.1653x over previous
import functools

import jax
import jax.numpy as jnp
from jax import lax
from jax.experimental import pallas as pl
from jax.experimental.pallas import tpu as pltpu
from jax.experimental.pallas import tpu_sc as plsc

B = 4096
D = 64
SEQ = 50
NC = 2
NS = 16
NW = NC * NS

NSPLIT = 2
GB = B // NSPLIT
IPW = GB // NW
UPW = B // NW
SPW = SEQ * GB // NW
SCHUNK = 640
NCHUNK = SPW // SCHUNK

BB = 256
NB1 = GB // BB
BB2 = 512
NB2 = B // BB2


_SC_PARAMS = pltpu.CompilerParams(use_tc_tiling_on_sc=False)


def _sc_user(user_pairs, uhalf):
    mesh = plsc.VectorSubcoreMesh(core_axis_name="c", subcore_axis_name="s")

    @functools.partial(
        pl.kernel,
        mesh=mesh,
        out_type=jax.ShapeDtypeStruct((B, 2 * D), jnp.float32),
        scratch_types=[
            pltpu.VMEM((UPW,), jnp.int32),
            pltpu.VMEM((UPW, 2 * D), jnp.float32),
            pltpu.SemaphoreType.DMA,
        ],
    )
    def k(tab, idx_hbm, out, idx_v, rows_v, sem):
        wid = lax.axis_index("s") * NC + lax.axis_index("c")
        base = wid * UPW
        pltpu.sync_copy(idx_hbm.at[pl.ds(base, UPW)], idx_v)
        pltpu.async_copy(tab.at[idx_v], rows_v, sem).wait()
        pltpu.sync_copy(rows_v, out.at[pl.ds(base, UPW)])

    return k(user_pairs, uhalf)


def _sc_group(item_table, iidx, sidx3):
    mesh = plsc.VectorSubcoreMesh(core_axis_name="c", subcore_axis_name="s")

    @functools.partial(
        pl.kernel,
        mesh=mesh,
        out_type=(
            jax.ShapeDtypeStruct((GB, D), jnp.float32),
            jax.ShapeDtypeStruct((SEQ * GB, D), jnp.float32),
        ),
        scratch_types=[
            pltpu.VMEM((IPW,), jnp.int32),
            pltpu.VMEM((IPW, D), jnp.float32),
            pltpu.VMEM((NCHUNK, SCHUNK), jnp.int32),
            pltpu.VMEM((SCHUNK, D), jnp.float32),
            pltpu.VMEM((SCHUNK, D), jnp.float32),
            pltpu.SemaphoreType.DMA,
            pltpu.SemaphoreType.DMA,
        ],
        compiler_params=_SC_PARAMS,
    )
    def k(tab, iref, sref, i_out, s_out,
          idx_v, rows_v, sidx_v, srows_a, srows_b, sem_a, sem_b):
        wid = lax.axis_index("s") * NC + lax.axis_index("c")
        base = wid * IPW
        pltpu.sync_copy(iref.at[pl.ds(base, IPW)], idx_v)
        pltpu.async_copy(tab.at[idx_v], rows_v, sem_a).wait()
        pltpu.sync_copy(rows_v, i_out.at[pl.ds(base, IPW)])
        pltpu.sync_copy(sref.at[wid], sidx_v)
        sbase = wid * SPW
        bufs = (srows_a, srows_b)
        sems = (sem_a, sem_b)
        copies = [
            pltpu.async_copy(tab.at[sidx_v.at[c]], bufs[c % 2], sems[c % 2])
            for c in range(2)
        ]
        for c in range(NCHUNK):
            copies[c % 2].wait()
            pltpu.sync_copy(bufs[c % 2],
                            s_out.at[pl.ds(sbase + c * SCHUNK, SCHUNK)])
            if c + 2 < NCHUNK:
                copies[c % 2] = pltpu.async_copy(
                    tab.at[sidx_v.at[c + 2]], bufs[c % 2], sems[c % 2])

    return k(item_table, iidx, sidx3)


def _dot(a, b):
    return lax.dot_general(a, b, (((1,), (0,)), ((), ())),
                           preferred_element_type=jnp.float32,
                           precision=lax.Precision.HIGHEST)


def _tc1_body(irows, seq3, maskf, g_i, a_i, c_i, cl_i, num, text,
              g_tab, a_tab, c_tab, cl_tab, pos,
              W_num, b_num, g_num, be_num,
              W_text, b_text, g_text, be_text,
              W_seq, b_seq, W_a1, b_a1, W_a2, b_a2,
              W_u1, b_u1, W_i1, b_i1, W_i2, b_i2,
              pu_ref, ivn_ref):
    relu = lambda x: jnp.maximum(x, 0.0)

    def onehot_embed(idx_ref, tab_ref, n):
        iot = lax.broadcasted_iota(jnp.int32, (BB, n), 1)
        oh = (iot == idx_ref[...]).astype(jnp.float32)
        return _dot(oh, tab_ref[...])

    g_e = onehot_embed(g_i, g_tab, 3)
    a_e = onehot_embed(a_i, a_tab, 10)
    c_e = onehot_embed(c_i, c_tab, 1000)
    cl_e = onehot_embed(cl_i, cl_tab, 100)

    num_proj = relu((_dot(num[...], W_num[...]) + b_num[...]) * g_num[...]
                    + be_num[...])
    text_proj = relu((_dot(text[...], W_text[...]) + b_text[...])
                     * g_text[...] + be_text[...])

    wa1 = W_a1[...]
    wa2 = W_a2[...]
    ba1 = b_a1[...]
    ba2 = b_a2[...]
    acc_sum = jnp.zeros((BB, D), jnp.float32)
    acc_max = jnp.full((BB, D), -jnp.inf, jnp.float32)
    m = jnp.full((BB, 1), -jnp.inf, jnp.float32)
    l = jnp.zeros((BB, 1), jnp.float32)
    acc_att = jnp.zeros((BB, D), jnp.float32)
    for s in range(SEQ):
        mc = maskf[:, s:s + 1]
        xs = (seq3[s] + pos[s:s + 1, :]) * mc
        acc_sum = acc_sum + xs
        acc_max = jnp.maximum(acc_max, xs)
        h = relu(_dot(xs, wa1) + ba1)
        lg = (_dot(h, wa2) + ba2) * mc - 1e9 * (1.0 - mc)
        nm = jnp.maximum(m, lg)
        sc = jnp.exp(m - nm)
        p = jnp.exp(lg - nm)
        l = l * sc + p
        acc_att = acc_att * sc + p * xs
        m = nm
    valid = jnp.sum(maskf[...], axis=1, keepdims=True)
    mean_p = acc_sum / (valid + 1e-8)
    att_p = acc_att / l

    ws = W_seq[...]
    seq_embed = relu(_dot(mean_p, ws[0:D]) + _dot(acc_max, ws[D:2 * D])
                     + _dot(att_p, ws[2 * D:3 * D]) + b_seq[...])

    wu1 = W_u1[...]
    pu = (_dot(g_e, wu1[D:2 * D]) + _dot(a_e, wu1[2 * D:3 * D])
          + _dot(c_e, wu1[3 * D:4 * D]) + _dot(cl_e, wu1[4 * D:5 * D])
          + _dot(num_proj, wu1[5 * D:6 * D])
          + _dot(seq_embed, wu1[6 * D:7 * D]) + b_u1[...])

    wi1 = W_i1[...]
    iv = relu(_dot(irows[...], wi1[0:D]) + _dot(text_proj, wi1[D:2 * D])
              + b_i1[...])
    iv = relu(_dot(iv, W_i2[...]) + b_i2[...])
    inn = jnp.maximum(jnp.sqrt(jnp.sum(iv * iv, axis=1, keepdims=True)),
                      1e-12)
    pu_ref[...] = pu
    ivn_ref[...] = iv / inn


def _tc2_body(upairs, par, pu, ivn, W_u1, W_u2, b_u2, out_ref):
    relu = lambda x: jnp.maximum(x, 0.0)
    p = par[...]
    urows = upairs[:, 0:D] * (1.0 - p) + upairs[:, D:2 * D] * p
    u = relu(pu[...] + _dot(urows, W_u1[0:D]))
    u2 = relu(_dot(u, W_u2[...]) + b_u2[...])
    un = jnp.maximum(jnp.sqrt(jnp.sum(u2 * u2, axis=1, keepdims=True)),
                     1e-12)
    out_ref[...] = jnp.sum(u2 * ivn[...], axis=1, keepdims=True) / un


def _row_spec(rows, cols):
    return pl.BlockSpec((rows, cols), lambda i: (i, 0))


def _full_spec(shape):
    nd = len(shape)
    return pl.BlockSpec(shape, lambda i: (0,) * nd)


def _tc1_kwargs():
    in_specs = [
        _row_spec(BB, D),
        pl.BlockSpec((SEQ, BB, D), lambda i: (0, i, 0)),
        _row_spec(BB, SEQ),
        _row_spec(BB, 1), _row_spec(BB, 1),
        _row_spec(BB, 1), _row_spec(BB, 1),
        _row_spec(BB, 16),
        _row_spec(BB, 128),
        _full_spec((3, D)), _full_spec((10, D)), _full_spec((1000, D)),
        _full_spec((100, D)), _full_spec((SEQ, D)),
        _full_spec((16, D)), _full_spec((1, D)), _full_spec((1, D)),
        _full_spec((1, D)),
        _full_spec((128, D)), _full_spec((1, D)), _full_spec((1, D)),
        _full_spec((1, D)),
        _full_spec((3 * D, D)), _full_spec((1, D)),
        _full_spec((D, D // 2)), _full_spec((1, D // 2)),
        _full_spec((D // 2, 1)), _full_spec((1, 1)),
        _full_spec((7 * D, 128)), _full_spec((1, 128)),
        _full_spec((2 * D, 128)), _full_spec((1, 128)),
        _full_spec((128, D)), _full_spec((1, D)),
    ]
    return dict(
        grid=(NB1,),
        in_specs=in_specs,
        out_specs=(_row_spec(BB, 128), _row_spec(BB, D)),
        out_shape=(jax.ShapeDtypeStruct((GB, 128), jnp.float32),
                   jax.ShapeDtypeStruct((GB, D), jnp.float32)),
        compiler_params=pltpu.CompilerParams(
            dimension_semantics=("arbitrary",)),
    )


def _tc2_kwargs():
    in_specs = [
        _row_spec(BB2, 2 * D),
        _row_spec(BB2, 1),
        _row_spec(BB2, 128),
        _row_spec(BB2, D),
        _full_spec((7 * D, 128)),
        _full_spec((128, D)),
        _full_spec((1, D)),
    ]
    return dict(
        grid=(NB2,),
        in_specs=in_specs,
        out_specs=_row_spec(BB2, 1),
        out_shape=jax.ShapeDtypeStruct((B, 1), jnp.float32),
        compiler_params=pltpu.CompilerParams(
            dimension_semantics=("arbitrary",)),
    )


def _group_args(g, i_rows, s_rows, sequence_mask, gender, age_range, city,
                cluster_id, user_numeric, item_text_feat, gender_table,
                age_table, city_table, cluster_table, position_table,
                W_num, b_num, g_num, be_num, W_text, b_text, g_text, be_text,
                W_seq, b_seq, W_a1, b_a1, W_a2, b_a2, W_u1, b_u1,
                W_i1, b_i1, W_i2, b_i2):
    r1 = lambda v: v.reshape(1, -1)
    k = 1.0 / jnp.sqrt(jnp.float32(1.0 + 1e-5))
    sl = slice(g * GB, (g + 1) * GB)
    col = lambda v: v[sl].astype(jnp.int32).reshape(GB, 1)
    return (
        i_rows, s_rows.reshape(SEQ, GB, D),
        sequence_mask[sl].astype(jnp.float32),
        col(gender), col(age_range), col(city), col(cluster_id),
        user_numeric[sl], item_text_feat[sl],
        gender_table, age_table, city_table, cluster_table, position_table,
        W_num, r1(b_num), r1(g_num) * k, r1(be_num),
        W_text, r1(b_text), r1(g_text) * k, r1(be_text),
        W_seq, r1(b_seq), W_a1, r1(b_a1), W_a2, r1(b_a2),
        W_u1, r1(b_u1), W_i1, r1(b_i1), W_i2, r1(b_i2),
    )


def kernel(user_id, item_id, gender, age_range, city, cluster_id,
           user_numeric, item_text_feat, user_sequence, sequence_mask,
           user_table, gender_table, age_table, city_table, cluster_table,
           item_table, position_table, W_num, b_num, g_num, be_num,
           W_text, b_text, g_text, be_text, W_seq, b_seq, W_a1, b_a1,
           W_a2, b_a2, W_u1, b_u1, W_u2, b_u2, W_i1, b_i1, W_i2, b_i2):
    uidx = user_id.astype(jnp.int32)
    iidx = item_id.astype(jnp.int32)
    nrows = user_table.shape[0]
    u_par = (uidx % 2).astype(jnp.float32).reshape(B, 1)
    pus, ivns = [], []
    for g in range(NSPLIT):
        sl = slice(g * GB, (g + 1) * GB)
        sidx3 = user_sequence[sl].astype(jnp.int32).T.reshape(
            NW, NCHUNK, SCHUNK)
        i_rows, s_rows = _sc_group(item_table, iidx[sl], sidx3)
        args = _group_args(g, i_rows, s_rows, sequence_mask, gender,
                           age_range, city, cluster_id, user_numeric,
                           item_text_feat, gender_table, age_table,
                           city_table, cluster_table, position_table,
                           W_num, b_num, g_num, be_num, W_text, b_text,
                           g_text, be_text, W_seq, b_seq, W_a1, b_a1,
                           W_a2, b_a2, W_u1, b_u1, W_i1, b_i1, W_i2, b_i2)
        pu, ivn = pl.pallas_call(_tc1_body, **_tc1_kwargs())(*args)
        pus.append(pu)
        ivns.append(ivn)
    pu = jnp.concatenate(pus, axis=0)
    ivn = jnp.concatenate(ivns, axis=0)
    dep = (pus[0][0, 0] * 0.0).astype(jnp.int32)
    u_pairs = _sc_user(user_table.reshape(nrows // 2, 2 * D),
                       uidx // 2 + dep)
    out = pl.pallas_call(_tc2_body, **_tc2_kwargs())(
        u_pairs, u_par, pu, ivn, W_u1, W_u2, b_u2.reshape(1, -1))
    return out.reshape(B)
